# baseline (speedup 1.0000x reference)
import functools

import jax
import jax.numpy as jnp
from jax import lax
from jax.experimental import pallas as pl
from jax.experimental.pallas import tpu as pltpu
from jax.experimental.pallas import tpu_sc as plsc

_B, _S, _D = 4096, 200, 64
_V = 1000000
_SCALE = float(_D) ** 0.5
_NC, _NS, _L = 2, 16, 16
_NW = _NC * _NS
_ROWS = _B * _S
_RPW = _ROWS // _NW
_C = 200
_NCHUNK = _RPW // _C
_BV = 16384


def _prep_body(wt_ref, o_ref):
    t = wt_ref[...].T * _SCALE
    o_ref[...] = jnp.concatenate([t, t], axis=1)


def _prep_table(weight_t):
    return pl.pallas_call(
        _prep_body,
        grid=(pl.cdiv(_V, _BV),),
        in_specs=[pl.BlockSpec((_D, _BV), lambda i: (0, i))],
        out_specs=pl.BlockSpec((_BV, 2 * _D), lambda i: (i, 0)),
        out_shape=jax.ShapeDtypeStruct((_V, 2 * _D), jnp.float32),
    )(weight_t)


def _emb_body(ids_hbm, table_hbm, out_hbm,
              idx_all, rows_a, rows_b, out_a, out_b, sem_a, sem_b,
              wsem_a, wsem_b):
    wid = lax.axis_index("s") * _NC + lax.axis_index("c")
    base = wid * _RPW
    pltpu.sync_copy(ids_hbm.at[pl.ds(base, _RPW)], idx_all)

    def gather(g, rows_v, sem):
        pltpu.async_copy(table_hbm.at[idx_all.at[pl.ds(g * _C, _C)]],
                         rows_v, sem)

    def wait_gather(g, rows_v, sem):
        pltpu.make_async_copy(table_hbm.at[idx_all.at[pl.ds(g * _C, _C)]],
                              rows_v, sem).wait()

    def drain(g, k, rows_v, out_v, wsem):
        off = base + g * _C

        @pl.when(k > 0)
        def _():
            pltpu.make_async_copy(
                out_v, out_hbm.at[pl.ds(off, _C)], wsem).wait()

        def extract_row(r2, c2):
            r = r2 * 2
            for rr in range(2):
                for j in range(_D // _L):
                    sl = pl.ds(j * _L, _L)
                    out_v[r + rr, sl] = rows_v[r + rr, sl]
            return c2

        lax.fori_loop(0, _C // 2, extract_row, 0)
        pltpu.async_copy(out_v, out_hbm.at[pl.ds(off, _C)], wsem)

    gather(0, rows_a, sem_a)

    def pair(k, carry):
        g0 = 2 * k
        gather(g0 + 1, rows_b, sem_b)
        wait_gather(g0, rows_a, sem_a)
        drain(g0, k, rows_a, out_a, wsem_a)

        @pl.when(k + 1 < _NCHUNK // 2)
        def _():
            gather(g0 + 2, rows_a, sem_a)

        wait_gather(g0 + 1, rows_b, sem_b)
        drain(g0 + 1, k, rows_b, out_b, wsem_b)
        return carry

    lax.fori_loop(0, _NCHUNK // 2, pair, 0)
    pltpu.make_async_copy(
        out_a, out_hbm.at[pl.ds(base + (_NCHUNK - 2) * _C, _C)],
        wsem_a).wait()
    pltpu.make_async_copy(
        out_b, out_hbm.at[pl.ds(base + (_NCHUNK - 1) * _C, _C)],
        wsem_b).wait()


def kernel(input_ids, weight):
    ids = input_ids.reshape(_ROWS)
    table = _prep_table(weight.T)
    mesh = plsc.VectorSubcoreMesh(core_axis_name="c", subcore_axis_name="s")
    run = functools.partial(
        pl.kernel,
        mesh=mesh,
        compiler_params=pltpu.CompilerParams(
            use_tc_tiling_on_sc=True, needs_layout_passes=False),
        out_type=jax.ShapeDtypeStruct((_ROWS, _D), jnp.float32),
        scratch_types=[
            pltpu.VMEM((_RPW,), jnp.int32),
            pltpu.VMEM((_C, 2 * _D), jnp.float32),
            pltpu.VMEM((_C, 2 * _D), jnp.float32),
            pltpu.VMEM((_C, _D), jnp.float32),
            pltpu.VMEM((_C, _D), jnp.float32),
            pltpu.SemaphoreType.DMA,
            pltpu.SemaphoreType.DMA,
            pltpu.SemaphoreType.DMA,
            pltpu.SemaphoreType.DMA,
        ],
    )(_emb_body)
    out = run(ids, table)
    return out.reshape(_B, _S, _D)


# speedup vs baseline: 1.0009x; 1.0009x over previous
import functools

import jax
import jax.numpy as jnp
from jax import lax
from jax.experimental import pallas as pl
from jax.experimental.pallas import tpu as pltpu
from jax.experimental.pallas import tpu_sc as plsc

_B, _S, _D = 4096, 200, 64
_V = 1000000
_SCALE = float(_D) ** 0.5
_NC, _NS, _L = 2, 16, 16
_NW = _NC * _NS
_ROWS = _B * _S
_RPW = _ROWS // _NW
_C = 256
_NCHUNK = _RPW // _C
_BV = 16384


def _prep_body(wt_ref, o_ref):
    t = wt_ref[...].T * _SCALE
    o_ref[...] = jnp.concatenate([t, t], axis=1)


def _prep_table(weight_t):
    return pl.pallas_call(
        _prep_body,
        grid=(pl.cdiv(_V, _BV),),
        in_specs=[pl.BlockSpec((_D, _BV), lambda i: (0, i))],
        out_specs=pl.BlockSpec((_BV, 2 * _D), lambda i: (i, 0)),
        out_shape=jax.ShapeDtypeStruct((_V, 2 * _D), jnp.float32),
    )(weight_t)


def _emb_body(ids_hbm, table_hbm, out_hbm,
              idx_a, idx_b, rows_a, rows_b, out_a,
              isem_a, isem_b, sem_a, sem_b, wsem_a):
    wid = lax.axis_index("s") * _NC + lax.axis_index("c")
    base = wid * _RPW

    def stage_idx(g, idx_v, isem):
        pltpu.async_copy(ids_hbm.at[pl.ds(base + g * _C, _C)], idx_v, isem)

    def wait_idx(g, idx_v, isem):
        pltpu.make_async_copy(ids_hbm.at[pl.ds(base + g * _C, _C)],
                              idx_v, isem).wait()

    def gather(g, idx_v, rows_v, sem):
        pltpu.async_copy(table_hbm.at[idx_v], rows_v, sem)

    def wait_gather(idx_v, rows_v, sem):
        pltpu.make_async_copy(table_hbm.at[idx_v], rows_v, sem).wait()

    def drain(g, rows_v, out_v, wsem):
        off = base + g * _C

        @pl.when(g > 0)
        def _():
            pltpu.make_async_copy(
                out_v, out_hbm.at[pl.ds(off, _C)], wsem).wait()

        def extract_row(r2, c2):
            r = r2 * 2
            for rr in range(2):
                for j in range(_D // _L):
                    sl = pl.ds(j * _L, _L)
                    out_v[r + rr, sl] = rows_v[r + rr, sl]
            return c2

        lax.fori_loop(0, _C // 2, extract_row, 0)
        pltpu.async_copy(out_v, out_hbm.at[pl.ds(off, _C)], wsem)

    stage_idx(0, idx_a, isem_a)
    stage_idx(1, idx_b, isem_b)
    wait_idx(0, idx_a, isem_a)
    gather(0, idx_a, rows_a, sem_a)

    def pair(k, carry):
        g0 = 2 * k
        wait_idx(g0 + 1, idx_b, isem_b)
        gather(g0 + 1, idx_b, rows_b, sem_b)
        wait_gather(idx_a, rows_a, sem_a)

        @pl.when(k + 1 < _NCHUNK // 2)
        def _():
            stage_idx(g0 + 2, idx_a, isem_a)

        drain(g0, rows_a, out_a, wsem_a)

        @pl.when(k + 1 < _NCHUNK // 2)
        def _():
            wait_idx(g0 + 2, idx_a, isem_a)
            gather(g0 + 2, idx_a, rows_a, sem_a)

        wait_gather(idx_b, rows_b, sem_b)

        @pl.when(k + 1 < _NCHUNK // 2)
        def _():
            stage_idx(g0 + 3, idx_b, isem_b)

        drain(g0 + 1, rows_b, out_a, wsem_a)
        return carry

    lax.fori_loop(0, _NCHUNK // 2, pair, 0)
    pltpu.make_async_copy(
        out_a, out_hbm.at[pl.ds(base + (_NCHUNK - 1) * _C, _C)],
        wsem_a).wait()


def kernel(input_ids, weight):
    ids = input_ids.reshape(_ROWS)
    table = _prep_table(weight.T)
    mesh = plsc.VectorSubcoreMesh(core_axis_name="c", subcore_axis_name="s")
    run = functools.partial(
        pl.kernel,
        mesh=mesh,
        compiler_params=pltpu.CompilerParams(
            use_tc_tiling_on_sc=True, needs_layout_passes=False),
        out_type=jax.ShapeDtypeStruct((_ROWS, _D), jnp.float32),
        scratch_types=[
            pltpu.VMEM((_C,), jnp.int32),
            pltpu.VMEM((_C,), jnp.int32),
            pltpu.VMEM((_C, 2 * _D), jnp.float32),
            pltpu.VMEM((_C, 2 * _D), jnp.float32),
            pltpu.VMEM((_C, _D), jnp.float32),
            pltpu.SemaphoreType.DMA,
            pltpu.SemaphoreType.DMA,
            pltpu.SemaphoreType.DMA,
            pltpu.SemaphoreType.DMA,
            pltpu.SemaphoreType.DMA,
        ],
    )(_emb_body)
    out = run(ids, table)
    return out.reshape(_B, _S, _D)


# speedup vs baseline: 1.0014x; 1.0005x over previous
import functools

import jax
import jax.numpy as jnp
from jax import lax
from jax.experimental import pallas as pl
from jax.experimental.pallas import tpu as pltpu
from jax.experimental.pallas import tpu_sc as plsc

_B, _S, _D = 4096, 200, 64
_V = 1000000
_SCALE = float(_D) ** 0.5
_NC, _NS, _L = 2, 16, 16
_NW = _NC * _NS
_ROWS = _B * _S
_RPW = _ROWS // _NW
_C = 256
_NCHUNK = _RPW // _C
_BV = 16384


def _prep_body(wt_ref, o_ref):
    eye = jnp.eye(_D, dtype=jnp.float32) * _SCALE
    t = jax.lax.dot_general(wt_ref[...], eye, (((0,), (0,)), ((), ())),
                            preferred_element_type=jnp.float32)
    o_ref[...] = jnp.concatenate([t, t], axis=1)


def _prep_table(weight_t):
    return pl.pallas_call(
        _prep_body,
        grid=(pl.cdiv(_V, _BV),),
        in_specs=[pl.BlockSpec((_D, _BV), lambda i: (0, i))],
        out_specs=pl.BlockSpec((_BV, 2 * _D), lambda i: (i, 0)),
        out_shape=jax.ShapeDtypeStruct((_V, 2 * _D), jnp.float32),
    )(weight_t)


def _emb_body(ids_hbm, table_hbm, out_hbm,
              idx_a, idx_b, rows_a, rows_b, out_a,
              isem_a, isem_b, sem_a, sem_b, wsem_a):
    wid = lax.axis_index("s") * _NC + lax.axis_index("c")
    base = wid * _RPW

    def stage_idx(g, idx_v, isem):
        pltpu.async_copy(ids_hbm.at[pl.ds(base + g * _C, _C)], idx_v, isem)

    def wait_idx(g, idx_v, isem):
        pltpu.make_async_copy(ids_hbm.at[pl.ds(base + g * _C, _C)],
                              idx_v, isem).wait()

    def gather(g, idx_v, rows_v, sem):
        pltpu.async_copy(table_hbm.at[idx_v], rows_v, sem)

    def wait_gather(idx_v, rows_v, sem):
        pltpu.make_async_copy(table_hbm.at[idx_v], rows_v, sem).wait()

    def drain(g, rows_v, out_v, wsem):
        off = base + g * _C

        @pl.when(g > 0)
        def _():
            pltpu.make_async_copy(
                out_v, out_hbm.at[pl.ds(off, _C)], wsem).wait()

        def extract_row(r2, c2):
            r = r2 * 2
            for rr in range(2):
                for j in range(_D // _L):
                    sl = pl.ds(j * _L, _L)
                    out_v[r + rr, sl] = rows_v[r + rr, sl]
            return c2

        lax.fori_loop(0, _C // 2, extract_row, 0)
        pltpu.async_copy(out_v, out_hbm.at[pl.ds(off, _C)], wsem)

    stage_idx(0, idx_a, isem_a)
    stage_idx(1, idx_b, isem_b)
    wait_idx(0, idx_a, isem_a)
    gather(0, idx_a, rows_a, sem_a)

    def pair(k, carry):
        g0 = 2 * k
        wait_idx(g0 + 1, idx_b, isem_b)
        gather(g0 + 1, idx_b, rows_b, sem_b)
        wait_gather(idx_a, rows_a, sem_a)

        @pl.when(k + 1 < _NCHUNK // 2)
        def _():
            stage_idx(g0 + 2, idx_a, isem_a)

        drain(g0, rows_a, out_a, wsem_a)

        @pl.when(k + 1 < _NCHUNK // 2)
        def _():
            wait_idx(g0 + 2, idx_a, isem_a)
            gather(g0 + 2, idx_a, rows_a, sem_a)

        wait_gather(idx_b, rows_b, sem_b)

        @pl.when(k + 1 < _NCHUNK // 2)
        def _():
            stage_idx(g0 + 3, idx_b, isem_b)

        drain(g0 + 1, rows_b, out_a, wsem_a)
        return carry

    lax.fori_loop(0, _NCHUNK // 2, pair, 0)
    pltpu.make_async_copy(
        out_a, out_hbm.at[pl.ds(base + (_NCHUNK - 1) * _C, _C)],
        wsem_a).wait()


def kernel(input_ids, weight):
    ids = input_ids.reshape(_ROWS)
    table = _prep_table(weight.T)
    mesh = plsc.VectorSubcoreMesh(core_axis_name="c", subcore_axis_name="s")
    run = functools.partial(
        pl.kernel,
        mesh=mesh,
        compiler_params=pltpu.CompilerParams(
            use_tc_tiling_on_sc=True, needs_layout_passes=False),
        out_type=jax.ShapeDtypeStruct((_ROWS, _D), jnp.float32),
        scratch_types=[
            pltpu.VMEM((_C,), jnp.int32),
            pltpu.VMEM((_C,), jnp.int32),
            pltpu.VMEM((_C, 2 * _D), jnp.float32),
            pltpu.VMEM((_C, 2 * _D), jnp.float32),
            pltpu.VMEM((_C, _D), jnp.float32),
            pltpu.SemaphoreType.DMA,
            pltpu.SemaphoreType.DMA,
            pltpu.SemaphoreType.DMA,
            pltpu.SemaphoreType.DMA,
            pltpu.SemaphoreType.DMA,
        ],
    )(_emb_body)
    out = run(ids, table)
    return out.reshape(_B, _S, _D)


# speedup vs baseline: 1.0053x; 1.0039x over previous
import functools

import jax
import jax.numpy as jnp
from jax import lax
from jax.experimental import pallas as pl
from jax.experimental.pallas import tpu as pltpu
from jax.experimental.pallas import tpu_sc as plsc

_B, _S, _D = 4096, 200, 64
_V = 1000000
_SCALE = float(_D) ** 0.5
_NC, _NS, _L = 2, 16, 16
_NW = _NC * _NS
_ROWS = _B * _S
_RPW = _ROWS // _NW
_C = 256
_NCHUNK = _RPW // _C
_BV = 16384


def _prep_body(wt_ref, o_ref):
    t = wt_ref[...].T * _SCALE
    o_ref[...] = jnp.concatenate([t, t], axis=1)


def _prep_table(weight_t):
    return pl.pallas_call(
        _prep_body,
        grid=(pl.cdiv(_V, _BV),),
        in_specs=[pl.BlockSpec((_D, _BV), lambda i: (0, i))],
        out_specs=pl.BlockSpec((_BV, 2 * _D), lambda i: (i, 0)),
        out_shape=jax.ShapeDtypeStruct((_V, 2 * _D), jnp.float32),
    )(weight_t)


def _emb_body(ids_hbm, table_hbm, out_hbm,
              idx_a, idx_b, rows_a, rows_b, out_a,
              isem_a, isem_b, sem_a, sem_b, wsem_a):
    wid = lax.axis_index("s") * _NC + lax.axis_index("c")
    base = wid * _RPW

    def stage_idx(g, idx_v, isem):
        pltpu.async_copy(ids_hbm.at[pl.ds(base + g * _C, _C)], idx_v, isem)

    def wait_idx(g, idx_v, isem):
        pltpu.make_async_copy(ids_hbm.at[pl.ds(base + g * _C, _C)],
                              idx_v, isem).wait()

    def gather(g, idx_v, rows_v, sem):
        pltpu.async_copy(table_hbm.at[idx_v], rows_v, sem)

    def wait_gather(idx_v, rows_v, sem):
        pltpu.make_async_copy(table_hbm.at[idx_v], rows_v, sem).wait()

    def drain(g, rows_v, out_v, wsem):
        off = base + g * _C

        @pl.when(g > 0)
        def _():
            pltpu.make_async_copy(
                out_v, out_hbm.at[pl.ds(off, _C)], wsem).wait()

        def extract_row(r2, c2):
            r = r2 * 2
            for rr in range(2):
                for j in range(_D // _L):
                    sl = pl.ds(j * _L, _L)
                    out_v[r + rr, sl] = rows_v[r + rr, sl]
            return c2

        lax.fori_loop(0, _C // 2, extract_row, 0)
        pltpu.async_copy(out_v, out_hbm.at[pl.ds(off, _C)], wsem)

    stage_idx(0, idx_a, isem_a)
    stage_idx(1, idx_b, isem_b)
    wait_idx(0, idx_a, isem_a)
    gather(0, idx_a, rows_a, sem_a)

    def pair(k, carry):
        g0 = 2 * k
        wait_idx(g0 + 1, idx_b, isem_b)
        gather(g0 + 1, idx_b, rows_b, sem_b)
        wait_gather(idx_a, rows_a, sem_a)

        @pl.when(k + 1 < _NCHUNK // 2)
        def _():
            stage_idx(g0 + 2, idx_a, isem_a)

        drain(g0, rows_a, out_a, wsem_a)

        @pl.when(k + 1 < _NCHUNK // 2)
        def _():
            wait_idx(g0 + 2, idx_a, isem_a)
            gather(g0 + 2, idx_a, rows_a, sem_a)

        wait_gather(idx_b, rows_b, sem_b)

        @pl.when(k + 1 < _NCHUNK // 2)
        def _():
            stage_idx(g0 + 3, idx_b, isem_b)

        drain(g0 + 1, rows_b, out_a, wsem_a)
        return carry

    lax.fori_loop(0, _NCHUNK // 2, pair, 0)
    pltpu.make_async_copy(
        out_a, out_hbm.at[pl.ds(base + (_NCHUNK - 1) * _C, _C)],
        wsem_a).wait()


def kernel(input_ids, weight):
    ids = input_ids.reshape(_ROWS)
    table = _prep_table(weight.T)
    mesh = plsc.VectorSubcoreMesh(core_axis_name="c", subcore_axis_name="s")
    run = functools.partial(
        pl.kernel,
        mesh=mesh,
        compiler_params=pltpu.CompilerParams(
            use_tc_tiling_on_sc=True, needs_layout_passes=False),
        out_type=jax.ShapeDtypeStruct((_ROWS, _D), jnp.float32),
        scratch_types=[
            pltpu.VMEM((_C,), jnp.int32),
            pltpu.VMEM((_C,), jnp.int32),
            pltpu.VMEM((_C, 2 * _D), jnp.float32),
            pltpu.VMEM((_C, 2 * _D), jnp.float32),
            pltpu.VMEM((_C, _D), jnp.float32),
            pltpu.SemaphoreType.DMA,
            pltpu.SemaphoreType.DMA,
            pltpu.SemaphoreType.DMA,
            pltpu.SemaphoreType.DMA,
            pltpu.SemaphoreType.DMA,
        ],
    )(_emb_body)
    out = run(ids, table)
    return out.reshape(_B, _S, _D)
